# Initial kernel scaffold; baseline (speedup 1.0000x reference)
import jax
import jax.numpy as jnp
from jax.experimental import pallas as pl


def kernel(pc, weights):
    raise NotImplementedError("write your pallas kernel here")


# speedup vs baseline: 5.4768x; 5.4768x over previous
import jax
import jax.numpy as jnp
from jax.experimental import pallas as pl

_K = 5
_ALPHA = 1.05
_BIG = 3.0e38

_R = 256
_CH = 8


def _nn_value_kernel(pc_ref, pct_ref, out_ref):
    j = pl.program_id(1)
    n = pc_ref.shape[1]
    rx = pct_ref[0, 0:1, pl.ds(j * _R, _R)]
    ry = pct_ref[0, 1:2, pl.ds(j * _R, _R)]
    rz = pct_ref[0, 2:3, pl.ds(j * _R, _R)]

    def body(i, carry):
        m1, m2, m3, m4, m5, m6 = carry
        cols = pc_ref[0, pl.ds(i * _CH, _CH), :]
        dx = cols[:, 0:1] - rx
        dy = cols[:, 1:2] - ry
        dz = cols[:, 2:3] - rz
        v = dx * dx + dy * dy + dz * dz
        r1 = jnp.minimum(m1, v)
        r2 = jnp.minimum(m2, jnp.maximum(m1, v))
        r3 = jnp.minimum(m3, jnp.maximum(m2, v))
        r4 = jnp.minimum(m4, jnp.maximum(m3, v))
        r5 = jnp.minimum(m5, jnp.maximum(m4, v))
        r6 = jnp.minimum(m6, jnp.maximum(m5, v))
        return r1, r2, r3, r4, r5, r6

    init = tuple(jnp.full((_CH, _R), _BIG, jnp.float32) for _ in range(6))
    m = jax.lax.fori_loop(0, n // _CH, body, init)
    cand = jnp.concatenate(m, axis=0)

    nslots = 6 * _CH
    iota = jax.lax.broadcasted_iota(jnp.int32, (nslots, _R), 0)
    work = cand
    ssum = jnp.zeros((1, _R), jnp.float32)
    m0 = None
    for t in range(6):
        mn = jnp.min(work, axis=0, keepdims=True)
        if t == 0:
            m0 = mn
        ssum = ssum + mn
        if t < 5:
            sel = jnp.where(work == mn, iota, nslots)
            amin = jnp.min(sel, axis=0, keepdims=True)
            work = jnp.where(iota == amin, _BIG, work)

    out_ref[0, 0:1, pl.ds(j * _R, _R)] = (ssum - m0) * (1.0 / _K)


def _loss_kernel(val_ref, w_ref, out_ref):
    b, _, n = val_ref.shape
    v = jnp.reshape(val_ref[...], (b, n))
    mean = jnp.sum(v, axis=1, keepdims=True) * (1.0 / n)
    d = v - mean
    var = jnp.sum(d * d, axis=1, keepdims=True) * (1.0 / (n - 1))
    thr = mean + _ALPHA * jnp.sqrt(var)
    masked = jnp.where(v > thr, v, 0.0)
    lb = jnp.sum(masked, axis=1, keepdims=True) * (1.0 / n)
    wl = lb * w_ref[...]
    out_ref[...] = jnp.sum(wl, axis=0, keepdims=True) * (1.0 / b)


def kernel(pc, weights):
    B, N, _ = pc.shape
    pc = pc.astype(jnp.float32)
    pct = jnp.transpose(pc, (0, 2, 1))
    nb = N // _R
    value = pl.pallas_call(
        _nn_value_kernel,
        grid=(B, nb),
        in_specs=[
            pl.BlockSpec((1, N, 3), lambda b, j: (b, 0, 0)),
            pl.BlockSpec((1, 3, N), lambda b, j: (b, 0, 0)),
        ],
        out_specs=pl.BlockSpec((1, 1, N), lambda b, j: (b, 0, 0)),
        out_shape=jax.ShapeDtypeStruct((B, 1, N), jnp.float32),
    )(pc, pct)
    out = pl.pallas_call(
        _loss_kernel,
        out_shape=jax.ShapeDtypeStruct((1, 1), jnp.float32),
    )(value, weights.astype(jnp.float32).reshape(B, 1))
    return out.reshape(())


# speedup vs baseline: 10.8242x; 1.9764x over previous
import jax
import jax.numpy as jnp
from jax.experimental import pallas as pl
from jax.experimental.pallas import tpu as pltpu

_K = 5
_ALPHA = 1.05
_BIG = 3.0e38

_R = 256
_CH = 32


def _nn_value_kernel(pc_ref, pct_ref, out_ref, cp_ref):
    n = pc_ref.shape[1]
    nb = n // _R
    ni = n // _CH

    cols = pc_ref[0]
    cp_ref[:, 0:3] = cols
    cp_ref[:, 3:4] = jnp.sum(cols * cols, axis=1, keepdims=True)

    def j_body(j, _):
        rows = pct_ref[0, :, pl.ds(j * _R, _R)]
        xxr = jnp.sum(rows * rows, axis=0, keepdims=True)
        rp = jnp.concatenate(
            [rows * (-2.0), jnp.ones((1, _R), jnp.float32)], axis=0)

        def mm(i):
            c = cp_ref[pl.ds(i * _CH, _CH), :]
            return jax.lax.dot_general(
                c, rp, (((1,), (0,)), ((), ())),
                preferred_element_type=jnp.float32,
                precision=jax.lax.Precision.HIGHEST)

        def i_body(i, carry):
            m1, m2, m3, m4, m5, m6, v = carry
            vn = mm(jnp.minimum(i + 1, ni - 1))
            r1 = jnp.minimum(m1, v)
            r2 = jnp.minimum(m2, jnp.maximum(m1, v))
            r3 = jnp.minimum(m3, jnp.maximum(m2, v))
            r4 = jnp.minimum(m4, jnp.maximum(m3, v))
            r5 = jnp.minimum(m5, jnp.maximum(m4, v))
            r6 = jnp.minimum(m6, jnp.maximum(m5, v))
            return r1, r2, r3, r4, r5, r6, vn

        init = tuple(jnp.full((_CH, _R), _BIG, jnp.float32) for _ in range(6))
        out = jax.lax.fori_loop(0, ni, i_body, init + (mm(0),))
        cand = jnp.concatenate(out[:6], axis=0)

        nslots = 6 * _CH
        iota = jax.lax.broadcasted_iota(jnp.int32, (nslots, _R), 0)
        work = cand
        ssum = jnp.zeros((1, _R), jnp.float32)
        m0 = None
        for t in range(6):
            mn = jnp.min(work, axis=0, keepdims=True)
            if t == 0:
                m0 = mn
            ssum = ssum + mn
            if t < 5:
                sel = jnp.where(work == mn, iota, nslots)
                amin = jnp.min(sel, axis=0, keepdims=True)
                work = jnp.where(iota == amin, _BIG, work)

        out_ref[0, 0:1, pl.ds(j * _R, _R)] = (ssum - m0) * (1.0 / _K) + xxr
        return 0

    jax.lax.fori_loop(0, nb, j_body, 0)


def _loss_kernel(val_ref, w_ref, out_ref):
    b, _, n = val_ref.shape
    v = jnp.reshape(val_ref[...], (b, n))
    mean = jnp.sum(v, axis=1, keepdims=True) * (1.0 / n)
    d = v - mean
    var = jnp.sum(d * d, axis=1, keepdims=True) * (1.0 / (n - 1))
    thr = mean + _ALPHA * jnp.sqrt(var)
    masked = jnp.where(v > thr, v, 0.0)
    lb = jnp.sum(masked, axis=1, keepdims=True) * (1.0 / n)
    wl = lb * w_ref[...]
    out_ref[...] = jnp.sum(wl, axis=0, keepdims=True) * (1.0 / b)


def kernel(pc, weights):
    B, N, _ = pc.shape
    pc = pc.astype(jnp.float32)
    pct = jnp.transpose(pc, (0, 2, 1))
    value = pl.pallas_call(
        _nn_value_kernel,
        grid=(B,),
        in_specs=[
            pl.BlockSpec((1, N, 3), lambda b: (b, 0, 0)),
            pl.BlockSpec((1, 3, N), lambda b: (b, 0, 0)),
        ],
        out_specs=pl.BlockSpec((1, 1, N), lambda b: (b, 0, 0)),
        out_shape=jax.ShapeDtypeStruct((B, 1, N), jnp.float32),
        scratch_shapes=[pltpu.VMEM((N, 4), jnp.float32)],
    )(pc, pct)
    out = pl.pallas_call(
        _loss_kernel,
        out_shape=jax.ShapeDtypeStruct((1, 1), jnp.float32),
    )(value, weights.astype(jnp.float32).reshape(B, 1))
    return out.reshape(())


# speedup vs baseline: 17.5461x; 1.6210x over previous
import jax
import jax.numpy as jnp
from jax.experimental import pallas as pl
from jax.experimental.pallas import tpu as pltpu

_K = 5
_ALPHA = 1.05
_BIG = 3.0e38

_R = 256
_CH = 8
_U = 8


def _nn_value_kernel(pc_ref, pct_ref, out_ref, cp_ref):
    n = pc_ref.shape[1]
    nb = n // _R
    ni = n // (_CH * _U)

    cols = pc_ref[0]
    cp_ref[:, 0:3] = cols
    cp_ref[:, 3:4] = jnp.sum(cols * cols, axis=1, keepdims=True)

    def j_body(j, _):
        rows = pct_ref[0, :, pl.ds(j * _R, _R)]
        xxr = jnp.sum(rows * rows, axis=0, keepdims=True)
        rp = jnp.concatenate(
            [rows * (-2.0), jnp.ones((1, _R), jnp.float32)], axis=0)

        def mm(i):
            c = cp_ref[pl.ds(i * _CH * _U, _CH * _U), :]
            return jax.lax.dot_general(
                c, rp, (((1,), (0,)), ((), ())),
                preferred_element_type=jnp.float32,
                precision=jax.lax.Precision.HIGHEST)

        def i_body(i, carry):
            m1, m2, m3, m4, m5, m6, v = carry
            vn = mm(jnp.minimum(i + 1, ni - 1))
            for u in range(_U):
                vu = jax.lax.slice_in_dim(v, u * _CH, (u + 1) * _CH, axis=0)
                m6 = jnp.minimum(m6, jnp.maximum(m5, vu))
                m5 = jnp.minimum(m5, jnp.maximum(m4, vu))
                m4 = jnp.minimum(m4, jnp.maximum(m3, vu))
                m3 = jnp.minimum(m3, jnp.maximum(m2, vu))
                m2 = jnp.minimum(m2, jnp.maximum(m1, vu))
                m1 = jnp.minimum(m1, vu)
            return m1, m2, m3, m4, m5, m6, vn

        init = tuple(jnp.full((_CH, _R), _BIG, jnp.float32) for _ in range(6))
        out = jax.lax.fori_loop(0, ni, i_body, init + (mm(0),))
        cand = jnp.concatenate(out[:6], axis=0)

        nslots = 6 * _CH
        iota = jax.lax.broadcasted_iota(jnp.int32, (nslots, _R), 0)
        work = cand
        ssum = jnp.zeros((1, _R), jnp.float32)
        m0 = None
        for t in range(6):
            mn = jnp.min(work, axis=0, keepdims=True)
            if t == 0:
                m0 = mn
            ssum = ssum + mn
            if t < 5:
                sel = jnp.where(work == mn, iota, nslots)
                amin = jnp.min(sel, axis=0, keepdims=True)
                work = jnp.where(iota == amin, _BIG, work)

        out_ref[0, 0:1, pl.ds(j * _R, _R)] = (ssum - m0) * (1.0 / _K) + xxr
        return 0

    jax.lax.fori_loop(0, nb, j_body, 0)


def _loss_kernel(val_ref, w_ref, out_ref):
    b, _, n = val_ref.shape
    v = jnp.reshape(val_ref[...], (b, n))
    mean = jnp.sum(v, axis=1, keepdims=True) * (1.0 / n)
    d = v - mean
    var = jnp.sum(d * d, axis=1, keepdims=True) * (1.0 / (n - 1))
    thr = mean + _ALPHA * jnp.sqrt(var)
    masked = jnp.where(v > thr, v, 0.0)
    lb = jnp.sum(masked, axis=1, keepdims=True) * (1.0 / n)
    wl = lb * w_ref[...]
    out_ref[...] = jnp.sum(wl, axis=0, keepdims=True) * (1.0 / b)


def kernel(pc, weights):
    B, N, _ = pc.shape
    pc = pc.astype(jnp.float32)
    pct = jnp.transpose(pc, (0, 2, 1))
    value = pl.pallas_call(
        _nn_value_kernel,
        grid=(B,),
        in_specs=[
            pl.BlockSpec((1, N, 3), lambda b: (b, 0, 0)),
            pl.BlockSpec((1, 3, N), lambda b: (b, 0, 0)),
        ],
        out_specs=pl.BlockSpec((1, 1, N), lambda b: (b, 0, 0)),
        out_shape=jax.ShapeDtypeStruct((B, 1, N), jnp.float32),
        scratch_shapes=[pltpu.VMEM((N, 4), jnp.float32)],
    )(pc, pct)
    out = pl.pallas_call(
        _loss_kernel,
        out_shape=jax.ShapeDtypeStruct((1, 1), jnp.float32),
    )(value, weights.astype(jnp.float32).reshape(B, 1))
    return out.reshape(())


# speedup vs baseline: 41.6989x; 2.3765x over previous
import jax
import jax.numpy as jnp
from jax.experimental import pallas as pl
from jax.experimental.pallas import tpu as pltpu

_K = 5
_ALPHA = 1.05
_BIG = 3.0e38

_R = 256
_CH = 8
_U = 8
_MC = 512


def _nn_value_kernel(pc_ref, pct_ref, out_ref, cp_ref, dp_ref):
    n = pc_ref.shape[1]
    nb = n // _R

    cols = pc_ref[0]
    cp_ref[:, 0:3] = cols
    cp_ref[:, 3:4] = jnp.sum(cols * cols, axis=1, keepdims=True)

    def j_body(j, _):
        rows = pct_ref[0, :, pl.ds(j * _R, _R)]
        xxr = jnp.sum(rows * rows, axis=0, keepdims=True)
        rp = jnp.concatenate(
            [rows * (-2.0), jnp.ones((1, _R), jnp.float32)], axis=0)

        for k in range(n // _MC):
            c = cp_ref[pl.ds(k * _MC, _MC), :]
            dp_ref[pl.ds(k * _MC, _MC), :] = jax.lax.dot_general(
                c, rp, (((1,), (0,)), ((), ())),
                preferred_element_type=jnp.float32,
                precision=jax.lax.Precision.HIGHEST)

        def ins(m, vu):
            m1, m2, m3, m4, m5, m6 = m
            m6 = jnp.minimum(m6, jnp.maximum(m5, vu))
            m5 = jnp.minimum(m5, jnp.maximum(m4, vu))
            m4 = jnp.minimum(m4, jnp.maximum(m3, vu))
            m3 = jnp.minimum(m3, jnp.maximum(m2, vu))
            m2 = jnp.minimum(m2, jnp.maximum(m1, vu))
            m1 = jnp.minimum(m1, vu)
            return m1, m2, m3, m4, m5, m6

        def i_body(i, carry):
            ma, mb = carry
            v = dp_ref[pl.ds(i * 2 * _U * _CH, 2 * _U * _CH), :]
            for u in range(_U):
                va = jax.lax.slice_in_dim(v, u * _CH, (u + 1) * _CH, axis=0)
                vb = jax.lax.slice_in_dim(
                    v, (_U + u) * _CH, (_U + u + 1) * _CH, axis=0)
                ma = ins(ma, va)
                mb = ins(mb, vb)
            return ma, mb

        init = tuple(jnp.full((_CH, _R), _BIG, jnp.float32) for _ in range(6))
        ma, mb = jax.lax.fori_loop(
            0, n // (2 * _U * _CH), i_body, (init, init))
        cand = jnp.concatenate(ma + mb, axis=0)

        nslots = 12 * _CH
        iota = jax.lax.broadcasted_iota(jnp.int32, (nslots, _R), 0)
        work = cand
        ssum = jnp.zeros((1, _R), jnp.float32)
        m0 = None
        for t in range(6):
            mn = jnp.min(work, axis=0, keepdims=True)
            if t == 0:
                m0 = mn
            ssum = ssum + mn
            if t < 5:
                sel = jnp.where(work == mn, iota, nslots)
                amin = jnp.min(sel, axis=0, keepdims=True)
                work = jnp.where(iota == amin, _BIG, work)

        out_ref[0, 0:1, pl.ds(j * _R, _R)] = (ssum - m0) * (1.0 / _K) + xxr
        return 0

    jax.lax.fori_loop(0, nb, j_body, 0)


def _loss_kernel(val_ref, w_ref, out_ref):
    b, _, n = val_ref.shape
    v = jnp.reshape(val_ref[...], (b, n))
    mean = jnp.sum(v, axis=1, keepdims=True) * (1.0 / n)
    d = v - mean
    var = jnp.sum(d * d, axis=1, keepdims=True) * (1.0 / (n - 1))
    thr = mean + _ALPHA * jnp.sqrt(var)
    masked = jnp.where(v > thr, v, 0.0)
    lb = jnp.sum(masked, axis=1, keepdims=True) * (1.0 / n)
    wl = lb * w_ref[...]
    out_ref[...] = jnp.sum(wl, axis=0, keepdims=True) * (1.0 / b)


def kernel(pc, weights):
    B, N, _ = pc.shape
    pc = pc.astype(jnp.float32)
    pct = jnp.transpose(pc, (0, 2, 1))
    value = pl.pallas_call(
        _nn_value_kernel,
        grid=(B,),
        in_specs=[
            pl.BlockSpec((1, N, 3), lambda b: (b, 0, 0)),
            pl.BlockSpec((1, 3, N), lambda b: (b, 0, 0)),
        ],
        out_specs=pl.BlockSpec((1, 1, N), lambda b: (b, 0, 0)),
        out_shape=jax.ShapeDtypeStruct((B, 1, N), jnp.float32),
        scratch_shapes=[pltpu.VMEM((N, 4), jnp.float32),
                        pltpu.VMEM((N, _R), jnp.float32)],
    )(pc, pct)
    out = pl.pallas_call(
        _loss_kernel,
        out_shape=jax.ShapeDtypeStruct((1, 1), jnp.float32),
    )(value, weights.astype(jnp.float32).reshape(B, 1))
    return out.reshape(())


# speedup vs baseline: 81.5094x; 1.9547x over previous
import jax
import jax.numpy as jnp
from jax.experimental import pallas as pl
from jax.experimental.pallas import tpu as pltpu

_K = 5
_ALPHA = 1.05
_BIG = 3.0e38

_R = 256
_CH = 8
_U = 8
_MC = 512


def _nn_value_kernel(pc_ref, pct_ref, out_ref, cp_ref, dp_ref):
    n = pc_ref.shape[1]
    nb = n // _R

    cols = pc_ref[0]
    cp_ref[:, 0:3] = cols
    cp_ref[:, 3:4] = jnp.sum(cols * cols, axis=1, keepdims=True)

    def j_body(j, _):
        rows = pct_ref[0, :, pl.ds(j * _R, _R)]
        xxr = jnp.sum(rows * rows, axis=0, keepdims=True)
        rp = jnp.concatenate(
            [rows * (-2.0), jnp.ones((1, _R), jnp.float32)], axis=0)

        for k in range(n // _MC):
            c = cp_ref[pl.ds(k * _MC, _MC), :]
            dp_ref[pl.ds(k * _MC, _MC), :] = jax.lax.dot_general(
                c, rp, (((1,), (0,)), ((), ())),
                preferred_element_type=jnp.float32,
                precision=jax.lax.Precision.DEFAULT)

        def ins(m, vu):
            m1, m2, m3, m4, m5, m6 = m
            m6 = jnp.minimum(m6, jnp.maximum(m5, vu))
            m5 = jnp.minimum(m5, jnp.maximum(m4, vu))
            m4 = jnp.minimum(m4, jnp.maximum(m3, vu))
            m3 = jnp.minimum(m3, jnp.maximum(m2, vu))
            m2 = jnp.minimum(m2, jnp.maximum(m1, vu))
            m1 = jnp.minimum(m1, vu)
            return m1, m2, m3, m4, m5, m6

        def i_body(i, carry):
            ma, mb = carry
            v = dp_ref[pl.ds(i * 2 * _U * _CH, 2 * _U * _CH), :]
            for u in range(_U):
                va = jax.lax.slice_in_dim(v, u * _CH, (u + 1) * _CH, axis=0)
                vb = jax.lax.slice_in_dim(
                    v, (_U + u) * _CH, (_U + u + 1) * _CH, axis=0)
                ma = ins(ma, va)
                mb = ins(mb, vb)
            return ma, mb

        init = tuple(jnp.full((_CH, _R), _BIG, jnp.float32) for _ in range(6))
        ma, mb = jax.lax.fori_loop(
            0, n // (2 * _U * _CH), i_body, (init, init))
        cand = jnp.concatenate(ma + mb, axis=0)

        nslots = 12 * _CH
        iota = jax.lax.broadcasted_iota(jnp.int32, (nslots, _R), 0)
        work = cand
        ssum = jnp.zeros((1, _R), jnp.float32)
        m0 = None
        for t in range(6):
            mn = jnp.min(work, axis=0, keepdims=True)
            if t == 0:
                m0 = mn
            ssum = ssum + mn
            if t < 5:
                sel = jnp.where(work == mn, iota, nslots)
                amin = jnp.min(sel, axis=0, keepdims=True)
                work = jnp.where(iota == amin, _BIG, work)

        out_ref[0, 0:1, pl.ds(j * _R, _R)] = (ssum - m0) * (1.0 / _K) + xxr
        return 0

    jax.lax.fori_loop(0, nb, j_body, 0)


def _loss_kernel(val_ref, w_ref, out_ref):
    b, _, n = val_ref.shape
    v = jnp.reshape(val_ref[...], (b, n))
    mean = jnp.sum(v, axis=1, keepdims=True) * (1.0 / n)
    d = v - mean
    var = jnp.sum(d * d, axis=1, keepdims=True) * (1.0 / (n - 1))
    thr = mean + _ALPHA * jnp.sqrt(var)
    masked = jnp.where(v > thr, v, 0.0)
    lb = jnp.sum(masked, axis=1, keepdims=True) * (1.0 / n)
    wl = lb * w_ref[...]
    out_ref[...] = jnp.sum(wl, axis=0, keepdims=True) * (1.0 / b)


def kernel(pc, weights):
    B, N, _ = pc.shape
    pc = pc.astype(jnp.float32)
    pct = jnp.transpose(pc, (0, 2, 1))
    value = pl.pallas_call(
        _nn_value_kernel,
        grid=(B,),
        in_specs=[
            pl.BlockSpec((1, N, 3), lambda b: (b, 0, 0)),
            pl.BlockSpec((1, 3, N), lambda b: (b, 0, 0)),
        ],
        out_specs=pl.BlockSpec((1, 1, N), lambda b: (b, 0, 0)),
        out_shape=jax.ShapeDtypeStruct((B, 1, N), jnp.float32),
        scratch_shapes=[pltpu.VMEM((N, 4), jnp.float32),
                        pltpu.VMEM((N, _R), jnp.float32)],
    )(pc, pct)
    out = pl.pallas_call(
        _loss_kernel,
        out_shape=jax.ShapeDtypeStruct((1, 1), jnp.float32),
    )(value, weights.astype(jnp.float32).reshape(B, 1))
    return out.reshape(())
